# baseline (speedup 1.0000x reference)
import jax
import jax.numpy as jnp
from jax.experimental import pallas as pl
from jax.experimental.pallas import tpu as pltpu


_N = 8 * 512 * 512
_STEPS = 16
_BLK = (1, 256, 512)


def _focal_block_sum(x, t):
    p = jax.nn.sigmoid(x)
    prob = jnp.where(t == 1, p, 1.0 - p)
    prob = jnp.clip(prob, 1e-8, 1.0 - 1e-8)
    one_m = 1.0 - prob
    return jnp.sum(one_m * one_m * (-jnp.log(prob)))


def _tc_body(x_ref, t_ref, o_ref):
    i = pl.program_id(0)
    s = _focal_block_sum(x_ref[...], t_ref[...])

    @pl.when(i == 0)
    def _():
        o_ref[0, 0] = s

    @pl.when(i > 0)
    def _():
        o_ref[0, 0] = o_ref[0, 0] + s

    @pl.when(i == _STEPS - 1)
    def _():
        o_ref[0, 0] = o_ref[0, 0] * (1.0 / _N)


def kernel(logit, target):
    x = logit
    t = target.astype(jnp.int32)
    out = pl.pallas_call(
        _tc_body,
        grid=(_STEPS,),
        in_specs=[
            pl.BlockSpec(_BLK, lambda i: (i // 2, i % 2, 0)),
            pl.BlockSpec(_BLK, lambda i: (i // 2, i % 2, 0)),
        ],
        out_specs=pl.BlockSpec(memory_space=pltpu.MemorySpace.SMEM),
        out_shape=jax.ShapeDtypeStruct((1, 1), jnp.float32),
        compiler_params=pltpu.CompilerParams(
            dimension_semantics=("arbitrary",),
        ),
    )(x, t)
    return out.reshape(())


# speedup vs baseline: 1.5632x; 1.5632x over previous
import jax
import jax.numpy as jnp
from jax.experimental import pallas as pl
from jax.experimental.pallas import tpu as pltpu


_N = 8 * 512 * 512
_STEPS = 4
_BLK = (2, 512, 512)


def _focal_block_sum(x, t):
    p = jax.nn.sigmoid(x)
    prob = jnp.where(t == 1, p, 1.0 - p)
    prob = jnp.clip(prob, 1e-8, 1.0 - 1e-8)
    one_m = 1.0 - prob
    return jnp.sum(one_m * one_m * (-jnp.log(prob)))


def _tc_body(x_ref, t_ref, o_ref):
    i = pl.program_id(0)
    s = _focal_block_sum(x_ref[...], t_ref[...])

    @pl.when(i == 0)
    def _():
        o_ref[0, 0] = s

    @pl.when(i > 0)
    def _():
        o_ref[0, 0] = o_ref[0, 0] + s

    @pl.when(i == _STEPS - 1)
    def _():
        o_ref[0, 0] = o_ref[0, 0] * (1.0 / _N)


def kernel(logit, target):
    x = logit
    t = target.astype(jnp.int32)
    out = pl.pallas_call(
        _tc_body,
        grid=(_STEPS,),
        in_specs=[
            pl.BlockSpec(_BLK, lambda i: (i, 0, 0)),
            pl.BlockSpec(_BLK, lambda i: (i, 0, 0)),
        ],
        out_specs=pl.BlockSpec(memory_space=pltpu.MemorySpace.SMEM),
        out_shape=jax.ShapeDtypeStruct((1, 1), jnp.float32),
        compiler_params=pltpu.CompilerParams(
            dimension_semantics=("arbitrary",),
        ),
    )(x, t)
    return out.reshape(())
